# baseline (speedup 1.0000x reference)
import functools

import jax
import jax.numpy as jnp
from jax import lax
from jax.experimental import pallas as pl
from jax.experimental.pallas import tpu as pltpu
from jax.experimental.pallas import tpu_sc as plsc

N = 10000
E = 320000
D = 128

NC = 2
NS = 16
L = 16

CHUNK = 80
EPS = E // (NC * NS)
NCHUNK = EPS // CHUNK
N_PAD = 10240
RPS = N_PAD // NS
ZROWS = 64
G = CHUNK // L
HALF = NCHUNK // 2


@functools.partial(
    pl.kernel,
    mesh=plsc.VectorSubcoreMesh(core_axis_name="c", subcore_axis_name="s"),
    out_type=jax.ShapeDtypeStruct((NC, N_PAD, D), jnp.float32),
    compiler_params=pltpu.CompilerParams(use_tc_tiling_on_sc=False,
                                         needs_layout_passes=False),
    scratch_types=[
        pltpu.VMEM((2, CHUNK), jnp.int32),
        pltpu.VMEM((2, CHUNK), jnp.int32),
        pltpu.VMEM((CHUNK,), jnp.float32),
        pltpu.VMEM((CHUNK,), jnp.float32),
        pltpu.VMEM((CHUNK,), jnp.int32),
        pltpu.VMEM((CHUNK,), jnp.int32),
        pltpu.VMEM((CHUNK,), jnp.int32),
        pltpu.VMEM((CHUNK,), jnp.int32),
        pltpu.VMEM((CHUNK, D), jnp.bfloat16),
        pltpu.VMEM((CHUNK, D), jnp.bfloat16),
        pltpu.VMEM((CHUNK, D), jnp.float32),
        pltpu.VMEM((CHUNK, D), jnp.float32),
        pltpu.VMEM_SHARED((N_PAD, D), jnp.float32),
        pltpu.VMEM((ZROWS, D), jnp.float32),
        pltpu.SemaphoreType.DMA,
        pltpu.SemaphoreType.DMA,
        pltpu.SemaphoreType.DMA,
        pltpu.SemaphoreType.DMA,
        pltpu.SemaphoreType.DMA,
        pltpu.SemaphoreType.DMA,
    ],
)
def _gcn_sc(idxT, vals2, feat, out,
            islab0, islab1, vslab0, vslab1, colb0, colb1, rowb0, rowb1,
            msgsb0, msgsb1, msgs0, msgs1, acc, zbuf,
            msem0, msem1, gsem0, gsem1, ssem0, ssem1):
    c = lax.axis_index("c")
    s = lax.axis_index("s")
    rbase = s * RPS
    ebase = (c * NS + s) * EPS

    islab = (islab0, islab1)
    vslab = (vslab0, vslab1)
    colb = (colb0, colb1)
    rowb = (rowb0, rowb1)
    msgsb = (msgsb0, msgsb1)
    msgs = (msgs0, msgs1)
    msem = (msem0, msem1)
    gsem = (gsem0, gsem1)
    ssem = (ssem0, ssem1)

    zero = jnp.zeros((L,), jnp.float32)
    splat_dnums = lax.GatherDimensionNumbers(
        offset_dims=(), collapsed_slice_dims=(0,), start_index_map=(0,))

    def lane_splat(vv, j):
        return lax.gather(vv, jnp.full((L, 1), j, jnp.int32), splat_dnums,
                          (1,), mode=lax.GatherScatterMode.PROMISE_IN_BOUNDS)

    @plsc.parallel_loop(0, ZROWS, unroll=4)
    def _(i):
        for q in range(D // L):
            zbuf[i, pl.ds(q * L, L)] = zero

    for i in range(RPS // ZROWS):
        pltpu.sync_copy(zbuf, acc.at[pl.ds(rbase + i * ZROWS, ZROWS), :])
    plsc.subcore_barrier()

    def start_meta(k, b):
        base = ebase + k * CHUNK
        pltpu.async_copy(idxT.at[:, pl.ds(base, CHUNK)], islab[b], msem[b])
        pltpu.async_copy(vals2.at[0, pl.ds(base, CHUNK)], vslab[b], msem[b])

    def wait_meta(b):
        pltpu.make_async_copy(idxT.at[:, pl.ds(0, CHUNK)], islab[b],
                              msem[b]).wait()
        pltpu.make_async_copy(vals2.at[0, pl.ds(0, CHUNK)], vslab[b],
                              msem[b]).wait()

    def index_compute(b):
        for g in range(G):
            seg = pl.ds(g * L, L)
            colb[b][seg] = islab[b][1, seg]
            rowb[b][seg] = islab[b][0, seg]

    def start_gather(b):
        pltpu.async_copy(feat.at[colb[b]], msgsb[b], gsem[b])

    def wait_gather(b):
        pltpu.make_async_copy(feat.at[colb[b]], msgsb[b], gsem[b]).wait()

    def start_scatter(b):
        pltpu.async_copy(msgs[b], acc.at[rowb[b]], ssem[b], add=True)

    def wait_scatter(b):
        pltpu.make_async_copy(msgs[b], acc.at[rowb[b]], ssem[b]).wait()

    def scale(b):
        @plsc.parallel_loop(0, G, unroll=2)
        def _(g):
            vv = vslab[b][pl.ds(g * L, L)]
            for j in range(L):
                sp = lane_splat(vv, j)
                je = g * L + j
                xs = []
                for h in range(D // (2 * L)):
                    packed = msgsb[b][je, pl.ds(h * 2 * L, 2 * L)]
                    a, bb = plsc.unpack(packed,
                                        format=plsc.PackFormat.INTERLEAVED)
                    xs += [a * sp, bb * sp]
                for q in range(D // L):
                    msgs[b][je, pl.ds(q * L, L)] = xs[q]

    start_meta(0, 0)
    start_meta(1, 1)
    wait_meta(0)
    index_compute(0)
    start_gather(0)

    def half_chunk(i, k, b, first):
        wait_meta(1 - b)

        @pl.when(jnp.logical_or(i >= 1, not first))
        def _():
            wait_scatter(1 - b)
        index_compute(1 - b)
        start_gather(1 - b)

        wait_gather(b)
        scale(b)

        if first:
            start_meta(k + 2, b)
        else:
            @pl.when(i < HALF - 1)
            def _():
                start_meta(k + 2, b)

        start_scatter(b)

    def pipe_step(i, carry):
        half_chunk(i, 2 * i, 0, True)
        half_chunk(i, 2 * i + 1, 1, False)
        return carry

    lax.fori_loop(0, HALF, pipe_step, 0)
    wait_gather(0)
    scale(0)
    start_scatter(0)
    wait_scatter(1)
    wait_scatter(0)
    plsc.subcore_barrier()

    pltpu.sync_copy(acc.at[pl.ds(rbase, RPS), :],
                    out.at[c, pl.ds(rbase, RPS), :])


def _combine_body(p_ref, w_ref, o_ref):
    o_ref[...] = jax.nn.relu((p_ref[0] + p_ref[1]) * w_ref[...])


_ROWS_BLK = 1024


@jax.jit
def _combine(partials, weight):
    return pl.pallas_call(
        _combine_body,
        grid=(N_PAD // _ROWS_BLK,),
        in_specs=[
            pl.BlockSpec((NC, _ROWS_BLK, D), lambda i: (0, i, 0)),
            pl.BlockSpec((1, D), lambda i: (0, 0)),
        ],
        out_specs=pl.BlockSpec((_ROWS_BLK, D), lambda i: (i, 0)),
        out_shape=jax.ShapeDtypeStruct((N_PAD, D), jnp.float32),
    )(partials, weight)


_PERM = []
for _h in range(D // 32):
    for _j in range(16):
        _PERM += [_h * 32 + _j, _h * 32 + 16 + _j]


def kernel(adj_indices, adj_values, features, weight):
    idxT = adj_indices.reshape(E, 2).T
    featb = features.astype(jnp.bfloat16)[:, jnp.array(_PERM, jnp.int32)]
    partials = _gcn_sc(idxT, adj_values, featb)
    return _combine(partials, weight)[:N]


# speedup vs baseline: 1.0100x; 1.0100x over previous
import functools

import jax
import jax.numpy as jnp
from jax import lax
from jax.experimental import pallas as pl
from jax.experimental.pallas import tpu as pltpu
from jax.experimental.pallas import tpu_sc as plsc

N = 10000
E = 320000
D = 128

NC = 2
NS = 16
L = 16

CHUNK = 80
EPS = E // (NC * NS)
NCHUNK = EPS // CHUNK
N_PAD = 10240
RPS = N_PAD // NS
ZROWS = 64
G = CHUNK // L
HALF = NCHUNK // 2


@functools.partial(
    pl.kernel,
    mesh=plsc.VectorSubcoreMesh(core_axis_name="c", subcore_axis_name="s"),
    out_type=jax.ShapeDtypeStruct((NC, N_PAD, D), jnp.float32),
    compiler_params=pltpu.CompilerParams(use_tc_tiling_on_sc=False,
                                         needs_layout_passes=False),
    scratch_types=[
        pltpu.VMEM((2, CHUNK), jnp.int32),
        pltpu.VMEM((2, CHUNK), jnp.int32),
        pltpu.VMEM((CHUNK,), jnp.float32),
        pltpu.VMEM((CHUNK,), jnp.float32),
        pltpu.VMEM((CHUNK,), jnp.int32),
        pltpu.VMEM((CHUNK,), jnp.int32),
        pltpu.VMEM((CHUNK,), jnp.int32),
        pltpu.VMEM((CHUNK,), jnp.int32),
        pltpu.VMEM((CHUNK, D), jnp.float32),
        pltpu.VMEM((CHUNK, D), jnp.float32),
        pltpu.VMEM_SHARED((N_PAD, D), jnp.float32),
        pltpu.VMEM((ZROWS, D), jnp.float32),
        pltpu.SemaphoreType.DMA,
        pltpu.SemaphoreType.DMA,
        pltpu.SemaphoreType.DMA,
        pltpu.SemaphoreType.DMA,
        pltpu.SemaphoreType.DMA,
        pltpu.SemaphoreType.DMA,
    ],
)
def _gcn_sc(idxT, vals2, feat, out,
            islab0, islab1, vslab0, vslab1, colb0, colb1, rowb0, rowb1,
            msgs0, msgs1, acc, zbuf,
            msem0, msem1, gsem0, gsem1, ssem0, ssem1):
    c = lax.axis_index("c")
    s = lax.axis_index("s")
    rbase = s * RPS
    ebase = (c * NS + s) * EPS

    islab = (islab0, islab1)
    vslab = (vslab0, vslab1)
    colb = (colb0, colb1)
    rowb = (rowb0, rowb1)
    msgs = (msgs0, msgs1)
    msem = (msem0, msem1)
    gsem = (gsem0, gsem1)
    ssem = (ssem0, ssem1)

    zero = jnp.zeros((L,), jnp.float32)
    splat_dnums = lax.GatherDimensionNumbers(
        offset_dims=(), collapsed_slice_dims=(0,), start_index_map=(0,))

    def lane_splat(vv, j):
        return lax.gather(vv, jnp.full((L, 1), j, jnp.int32), splat_dnums,
                          (1,), mode=lax.GatherScatterMode.PROMISE_IN_BOUNDS)

    @plsc.parallel_loop(0, ZROWS, unroll=4)
    def _(i):
        for q in range(D // L):
            zbuf[i, pl.ds(q * L, L)] = zero

    for i in range(RPS // ZROWS):
        pltpu.sync_copy(zbuf, acc.at[pl.ds(rbase + i * ZROWS, ZROWS), :])
    plsc.subcore_barrier()

    def start_meta(k, b):
        base = ebase + k * CHUNK
        pltpu.async_copy(idxT.at[:, pl.ds(base, CHUNK)], islab[b], msem[b])
        pltpu.async_copy(vals2.at[0, pl.ds(base, CHUNK)], vslab[b], msem[b])

    def wait_meta(b):
        pltpu.make_async_copy(idxT.at[:, pl.ds(0, CHUNK)], islab[b],
                              msem[b]).wait()
        pltpu.make_async_copy(vals2.at[0, pl.ds(0, CHUNK)], vslab[b],
                              msem[b]).wait()

    def index_compute(b):
        for g in range(G):
            seg = pl.ds(g * L, L)
            colb[b][seg] = islab[b][1, seg]
            rowb[b][seg] = islab[b][0, seg]

    def start_gather(b):
        pltpu.async_copy(feat.at[colb[b]], msgs[b], gsem[b])

    def wait_gather(b):
        pltpu.make_async_copy(feat.at[colb[b]], msgs[b], gsem[b]).wait()

    def start_scatter(b):
        pltpu.async_copy(msgs[b], acc.at[rowb[b]], ssem[b], add=True)

    def wait_scatter(b):
        pltpu.make_async_copy(msgs[b], acc.at[rowb[b]], ssem[b]).wait()

    def scale(b):
        @plsc.parallel_loop(0, G, unroll=2)
        def _(g):
            vv = vslab[b][pl.ds(g * L, L)]
            for j in range(L):
                sp = lane_splat(vv, j)
                je = g * L + j
                xs = [msgs[b][je, pl.ds(q * L, L)] * sp for q in range(D // L)]
                for q in range(D // L):
                    msgs[b][je, pl.ds(q * L, L)] = xs[q]

    start_meta(0, 0)
    start_meta(1, 1)
    wait_meta(0)
    index_compute(0)
    start_gather(0)

    def half_chunk(i, k, b, first):
        wait_meta(1 - b)

        @pl.when(jnp.logical_or(i >= 1, not first))
        def _():
            wait_scatter(1 - b)
        index_compute(1 - b)
        start_gather(1 - b)

        wait_gather(b)
        scale(b)

        if first:
            start_meta(k + 2, b)
        else:
            @pl.when(i < HALF - 1)
            def _():
                start_meta(k + 2, b)

        start_scatter(b)

    def pipe_step(i, carry):
        half_chunk(i, 2 * i, 0, True)
        half_chunk(i, 2 * i + 1, 1, False)
        return carry

    lax.fori_loop(0, HALF, pipe_step, 0)
    wait_gather(0)
    scale(0)
    start_scatter(0)
    wait_scatter(1)
    wait_scatter(0)
    plsc.subcore_barrier()

    pltpu.sync_copy(acc.at[pl.ds(rbase, RPS), :],
                    out.at[c, pl.ds(rbase, RPS), :])


def _combine_body(p_ref, w_ref, o_ref):
    o_ref[...] = jax.nn.relu((p_ref[0] + p_ref[1]) * w_ref[...])


_ROWS_BLK = 1024


@jax.jit
def _combine(partials, weight):
    return pl.pallas_call(
        _combine_body,
        grid=(N_PAD // _ROWS_BLK,),
        in_specs=[
            pl.BlockSpec((NC, _ROWS_BLK, D), lambda i: (0, i, 0)),
            pl.BlockSpec((1, D), lambda i: (0, 0)),
        ],
        out_specs=pl.BlockSpec((_ROWS_BLK, D), lambda i: (i, 0)),
        out_shape=jax.ShapeDtypeStruct((N_PAD, D), jnp.float32),
    )(partials, weight)


def kernel(adj_indices, adj_values, features, weight):
    idxT = adj_indices.reshape(E, 2).T
    partials = _gcn_sc(idxT, adj_values, features)
    return _combine(partials, weight)[:N]
